# baseline (speedup 1.0000x reference)
import jax
import jax.numpy as jnp
from jax import lax
from jax.experimental import pallas as pl
from jax.experimental.pallas import tpu as pltpu
from jax.experimental.pallas import tpu_sc as plsc

B = 16384
D = 16
F = 26

NC = 2
NS = 16
NW = NC * NS
BPW = B // NW
CH = 128
NCH = BPW // CH


def _gather_body(cat, table, out, idx_v, emb, sem):
    wid = lax.axis_index("s") * NC + lax.axis_index("c")
    base = wid * BPW

    pltpu.sync_copy(cat.at[pl.ds(base, BPW)], idx_v)
    ones = jnp.ones((16,), jnp.int32)

    @pl.loop(0, BPW // 16)
    def shift(i):
        idx_v[pl.ds(i * 16, 16)] = idx_v[pl.ds(i * 16, 16)] + ones

    copies = []
    for c in range(NCH):
        off = pl.multiple_of(c * CH, CH)
        copies.append(pltpu.async_copy(
            table.at[idx_v.at[pl.ds(off, CH)]],
            emb.at[pl.ds(off, CH), :],
            sem,
        ))
    for cp in copies:
        cp.wait()

    pltpu.sync_copy(emb, out.at[pl.ds(base, BPW), :])


_gather = pl.kernel(
    _gather_body,
    out_type=jax.ShapeDtypeStruct((B, D), jnp.float32),
    mesh=plsc.VectorSubcoreMesh(
        core_axis_name="c", subcore_axis_name="s",
        num_cores=NC, num_subcores=NS,
    ),
    scratch_types=[
        pltpu.VMEM((BPW,), jnp.int32),
        pltpu.VMEM((BPW, D), jnp.float32),
        pltpu.SemaphoreType.DMA,
    ],
    compiler_params=pltpu.CompilerParams(use_tc_tiling_on_sc=False),
)


@jax.jit
def _run(numeric, cats, tables):
    blocks = [_gather(cats[t], tables[t]) for t in range(F)]
    return jnp.concatenate([numeric] + blocks, axis=-1)


def kernel(numeric, cat_0, cat_1, cat_2, cat_3, cat_4, cat_5, cat_6, cat_7, cat_8, cat_9, cat_10, cat_11, cat_12, cat_13, cat_14, cat_15, cat_16, cat_17, cat_18, cat_19, cat_20, cat_21, cat_22, cat_23, cat_24, cat_25, W_0, W_1, W_2, W_3, W_4, W_5, W_6, W_7, W_8, W_9, W_10, W_11, W_12, W_13, W_14, W_15, W_16, W_17, W_18, W_19, W_20, W_21, W_22, W_23, W_24, W_25):
    cats = (cat_0, cat_1, cat_2, cat_3, cat_4, cat_5, cat_6, cat_7, cat_8,
            cat_9, cat_10, cat_11, cat_12, cat_13, cat_14, cat_15, cat_16,
            cat_17, cat_18, cat_19, cat_20, cat_21, cat_22, cat_23, cat_24,
            cat_25)
    tables = (W_0, W_1, W_2, W_3, W_4, W_5, W_6, W_7, W_8, W_9, W_10, W_11,
              W_12, W_13, W_14, W_15, W_16, W_17, W_18, W_19, W_20, W_21,
              W_22, W_23, W_24, W_25)
    return _run(numeric, cats, tables)
